# baseline (speedup 1.0000x reference)
import functools
import jax
import jax.numpy as jnp
from jax import lax
from jax.experimental import pallas as pl
from jax.experimental.pallas import tpu as pltpu
from jax.experimental.pallas import tpu_sc as plsc

_SC_WORKERS = 32


def _pick_chunk(per_w, D):
    ch = per_w
    while ch * D * 4 > 280_000 and ch > 8:
        ch //= 2
    return ch


def _sc_gather(table, idx):
    R, D = table.shape
    assert D % 8 == 0, "row width must be a multiple of 8 words (HBM row stride)"
    Q = idx.shape[0]
    per_w = Q // _SC_WORKERS
    ch = _pick_chunk(per_w, D)
    nchunk = per_w // ch
    mesh = plsc.VectorSubcoreMesh(core_axis_name="c", subcore_axis_name="s")

    @functools.partial(
        pl.kernel,
        out_type=jax.ShapeDtypeStruct((Q, D), jnp.float32),
        mesh=mesh,
        compiler_params=pltpu.CompilerParams(use_tc_tiling_on_sc=False),
        scratch_types=[
            pltpu.VMEM((ch,), jnp.int32),
            pltpu.VMEM((ch, D), jnp.float32),
            pltpu.SemaphoreType.DMA,
        ],
    )
    def gk(table_hbm, idx_hbm, out_hbm, idx_v, rows_v, sem):
        wid = lax.axis_index("s") * 2 + lax.axis_index("c")
        base = wid * per_w

        def body(i, carry):
            off = base + i * ch
            pltpu.sync_copy(idx_hbm.at[pl.ds(off, ch)], idx_v)
            pltpu.async_copy(table_hbm.at[idx_v], rows_v, sem).wait()
            pltpu.sync_copy(rows_v, out_hbm.at[pl.ds(off, ch)])
            return carry

        lax.fori_loop(0, nchunk, body, 0)

    return gk(table, idx)


def _fps_kernel(p_ref, out_ref, *, M):
    B, N = p_ref.shape[1], p_ref.shape[2]
    iota = lax.broadcasted_iota(jnp.int32, (B, N), 1)
    iota_m = lax.broadcasted_iota(jnp.int32, (1, 1, M), 2)

    def body(t, carry):
        dists, c = carry
        out_ref[...] = jnp.where(iota_m == t, c, out_ref[...])
        p = p_ref[...]
        d3 = (p - c) ** 2
        d = d3[0] + d3[1] + d3[2]
        dists = jnp.minimum(dists, d)
        m = jnp.max(dists, axis=1, keepdims=True)
        sel = jnp.where(dists == m, iota, N)
        far = jnp.min(sel, axis=1, keepdims=True)
        mask = (iota == far)[None]
        c_new = jnp.max(jnp.where(mask, p, -1e37), axis=2, keepdims=True)
        return dists, c_new

    dists0 = jnp.full((B, N), 1e10, jnp.float32)
    c0 = p_ref[:, :, 0:1]
    lax.fori_loop(0, M, body, (dists0, c0))


def _fps_pallas(planes, M):
    _, B, N = planes.shape
    return pl.pallas_call(
        functools.partial(_fps_kernel, M=M),
        out_shape=jax.ShapeDtypeStruct((3, B, M), jnp.float32),
    )(planes)


def _sqdist(a, b):
    aa = jnp.sum(a * a, axis=-1)[:, :, None]
    bb = jnp.sum(b * b, axis=-1)[:, None, :]
    ab = jnp.einsum('bmd,bnd->bmn', a, b)
    return aa + bb - 2.0 * ab


def _fps(xyz, npoint):
    xyz = jax.lax.stop_gradient(xyz)
    B, N, _ = xyz.shape

    def step(carry, _):
        dists, far = carry
        centroid = jnp.take_along_axis(xyz, far[:, None, None], axis=1)
        d = jnp.sum((xyz - centroid) ** 2, axis=-1)
        dists = jnp.minimum(dists, d)
        nxt = jnp.argmax(dists, axis=-1).astype(jnp.int32)
        return (dists, nxt), far

    dists0 = jnp.full((B, N), 1e10, dtype=xyz.dtype)
    far0 = jnp.zeros((B,), dtype=jnp.int32)
    _, idxs = jax.lax.scan(step, (dists0, far0), None, length=npoint)
    return jnp.transpose(idxs, (1, 0))


def _bgather(x, idx):
    return jax.vmap(lambda xi, ii: xi[ii])(x, idx)


def _knn(q, ref, k):
    d = _sqdist(q, ref)
    _, idx = jax.lax.top_k(-d, k)
    return idx


def _head_kernel(pooled_ref, wc1_ref, wc2_ref, wc3_ref, out_ref):
    x = pooled_ref[...]
    x = jax.nn.relu(jnp.dot(x, wc1_ref[...], preferred_element_type=jnp.float32))
    x = jax.nn.relu(jnp.dot(x, wc2_ref[...], preferred_element_type=jnp.float32))
    out_ref[...] = jnp.dot(x, wc3_ref[...], preferred_element_type=jnp.float32)


def _classifier_head(pooled, Wc1, Wc2, Wc3):
    B = pooled.shape[0]
    return pl.pallas_call(
        _head_kernel,
        out_shape=jax.ShapeDtypeStruct((B, Wc3.shape[1]), jnp.float32),
    )(pooled, Wc1, Wc2, Wc3)


def kernel(xyz, feature, W_embed, Wt0, Wt1, Wt2, Wt3, Wb0, Wb1, Wb2, Wb3, Wc1, Wc2, Wc3):
    feat = jnp.transpose(feature, (0, 2, 1))
    f = jax.nn.relu(jnp.einsum('bnc,co->bno', feat, W_embed))
    cur_xyz = xyz
    cur_planes = jnp.transpose(xyz, (2, 0, 1))
    k = 32
    for Wt, Wb in zip((Wt0, Wt1, Wt2, Wt3), (Wb0, Wb1, Wb2, Wb3)):
        B, N, d = f.shape
        M = N // 2
        new_planes = _fps_pallas(cur_planes, M)
        new_xyz = jnp.transpose(new_planes, (1, 2, 0))
        nidx = _knn(new_xyz, cur_xyz, k)
        dpad = -(d + 3) % 8
        table = jnp.concatenate(
            [f, cur_xyz] + ([jnp.zeros((B, N, dpad), jnp.float32)] if dpad else []),
            axis=-1).reshape(B * N, d + 3 + dpad)
        nidx_glob = (nidx + (jnp.arange(B, dtype=jnp.int32) * N)[:, None, None]).reshape(-1)
        rows = _sc_gather(table, nidx_glob).reshape(B, M, k, d + 3 + dpad)
        g_feat = rows[..., :d]
        g_xyz = rows[..., d:d + 3]
        rel = g_xyz - new_xyz[:, :, None, :]
        std = jnp.std(rel, axis=(2, 3), keepdims=True) + 1e-5
        rel = rel / std
        g = jnp.concatenate([g_feat, rel], axis=-1)
        h = jax.nn.relu(jnp.einsum('bmkc,co->bmko', g, Wt))
        h = jnp.max(h, axis=2)
        h = jax.nn.relu(h + jax.nn.relu(jnp.einsum('bmc,co->bmo', h, Wb)))
        cur_xyz, cur_planes, f = new_xyz, new_planes, h
    pooled = jnp.max(f, axis=1)
    return _classifier_head(pooled, Wc1, Wc2, Wc3)


# speedup vs baseline: 1.0682x; 1.0682x over previous
import functools
import jax
import jax.numpy as jnp
from jax import lax
from jax.experimental import pallas as pl
from jax.experimental.pallas import tpu as pltpu
from jax.experimental.pallas import tpu_sc as plsc

_SC_WORKERS = 32


def _pick_chunk(per_w, D):
    ch = per_w
    while ch * D * 4 > 280_000 and ch > 8:
        ch //= 2
    return ch


def _sc_gather(table, idx):
    R, D = table.shape
    assert D % 8 == 0, "row width must be a multiple of 8 words (HBM row stride)"
    Q = idx.shape[0]
    per_w = Q // _SC_WORKERS
    ch = _pick_chunk(per_w, D)
    nchunk = per_w // ch
    mesh = plsc.VectorSubcoreMesh(core_axis_name="c", subcore_axis_name="s")

    @functools.partial(
        pl.kernel,
        out_type=jax.ShapeDtypeStruct((Q, D), jnp.float32),
        mesh=mesh,
        compiler_params=pltpu.CompilerParams(use_tc_tiling_on_sc=False),
        scratch_types=[
            pltpu.VMEM((ch,), jnp.int32),
            pltpu.VMEM((ch, D), jnp.float32),
            pltpu.SemaphoreType.DMA,
        ],
    )
    def gk(table_hbm, idx_hbm, out_hbm, idx_v, rows_v, sem):
        wid = lax.axis_index("s") * 2 + lax.axis_index("c")
        base = wid * per_w

        def body(i, carry):
            off = base + i * ch
            pltpu.sync_copy(idx_hbm.at[pl.ds(off, ch)], idx_v)
            pltpu.async_copy(table_hbm.at[idx_v], rows_v, sem).wait()
            pltpu.sync_copy(rows_v, out_hbm.at[pl.ds(off, ch)])
            return carry

        lax.fori_loop(0, nchunk, body, 0)

    return gk(table, idx)


def _fps_kernel(p_ref, out_ref, *, M):
    B, N = p_ref.shape[1], p_ref.shape[2]
    iota = lax.broadcasted_iota(jnp.int32, (B, N), 1)
    iota_m = lax.broadcasted_iota(jnp.int32, (1, 1, M), 2)

    def body(t, carry):
        dists, c = carry
        out_ref[...] = jnp.where(iota_m == t, c, out_ref[...])
        p = p_ref[...]
        d3 = (p - c) ** 2
        d = d3[0] + d3[1] + d3[2]
        dists = jnp.minimum(dists, d)
        m = jnp.max(dists, axis=1, keepdims=True)
        sel = jnp.where(dists == m, iota, N)
        far = jnp.min(sel, axis=1, keepdims=True)
        mask = (iota == far)[None]
        c_new = jnp.max(jnp.where(mask, p, -1e37), axis=2, keepdims=True)
        return dists, c_new

    dists0 = jnp.full((B, N), 1e10, jnp.float32)
    c0 = p_ref[:, :, 0:1]
    lax.fori_loop(0, M, body, (dists0, c0))


def _fps_pallas(planes, M):
    _, B, N = planes.shape
    return pl.pallas_call(
        functools.partial(_fps_kernel, M=M),
        out_shape=jax.ShapeDtypeStruct((3, B, M), jnp.float32),
    )(planes)


def _knn_kernel(q_ref, r_ref, out_ref, *, k):
    q = q_ref[0]
    r = r_ref[0]
    Mt = q.shape[0]
    N = r.shape[1]
    aa = jnp.sum(q * q, axis=1, keepdims=True)
    bb = jnp.sum(r * r, axis=0, keepdims=True)
    ab = lax.dot_general(q, r, (((1,), (0,)), ((), ())),
                         preferred_element_type=jnp.float32)
    d = aa + bb - 2.0 * ab
    iota = lax.broadcasted_iota(jnp.int32, (Mt, N), 1)
    iota_k = lax.broadcasted_iota(jnp.int32, (Mt, k), 1)

    def round_fn(j, carry):
        d, acc = carry
        m = jnp.min(d, axis=1, keepdims=True)
        idx = jnp.min(jnp.where(d == m, iota, N), axis=1, keepdims=True)
        acc = jnp.where(iota_k == j, idx, acc)
        d = jnp.where(iota == idx, jnp.inf, d)
        return d, acc

    _, acc = lax.fori_loop(0, k, round_fn,
                           (d, jnp.zeros((Mt, k), jnp.int32)))
    out_ref[0] = acc


def _knn_pallas(new_xyz, ref_bplanes, k):
    B, M, _ = new_xyz.shape
    N = ref_bplanes.shape[2]
    Mt = min(128, M)
    return pl.pallas_call(
        functools.partial(_knn_kernel, k=k),
        grid=(B, M // Mt),
        in_specs=[pl.BlockSpec((1, Mt, 3), lambda b, m: (b, m, 0)),
                  pl.BlockSpec((1, 3, N), lambda b, m: (b, 0, 0))],
        out_specs=pl.BlockSpec((1, Mt, k), lambda b, m: (b, m, 0)),
        out_shape=jax.ShapeDtypeStruct((B, M, k), jnp.int32),
    )(new_xyz, ref_bplanes)


def _head_kernel(pooled_ref, wc1_ref, wc2_ref, wc3_ref, out_ref):
    x = pooled_ref[...]
    x = jax.nn.relu(jnp.dot(x, wc1_ref[...], preferred_element_type=jnp.float32))
    x = jax.nn.relu(jnp.dot(x, wc2_ref[...], preferred_element_type=jnp.float32))
    out_ref[...] = jnp.dot(x, wc3_ref[...], preferred_element_type=jnp.float32)


def _classifier_head(pooled, Wc1, Wc2, Wc3):
    B = pooled.shape[0]
    return pl.pallas_call(
        _head_kernel,
        out_shape=jax.ShapeDtypeStruct((B, Wc3.shape[1]), jnp.float32),
    )(pooled, Wc1, Wc2, Wc3)


def kernel(xyz, feature, W_embed, Wt0, Wt1, Wt2, Wt3, Wb0, Wb1, Wb2, Wb3, Wc1, Wc2, Wc3):
    feat = jnp.transpose(feature, (0, 2, 1))
    f = jax.nn.relu(jnp.einsum('bnc,co->bno', feat, W_embed))
    cur_xyz = xyz
    cur_planes = jnp.transpose(xyz, (2, 0, 1))
    k = 32
    for Wt, Wb in zip((Wt0, Wt1, Wt2, Wt3), (Wb0, Wb1, Wb2, Wb3)):
        B, N, d = f.shape
        M = N // 2
        new_planes = _fps_pallas(cur_planes, M)
        new_xyz = jnp.transpose(new_planes, (1, 2, 0))
        nidx = _knn_pallas(new_xyz, jnp.transpose(cur_planes, (1, 0, 2)), k)
        dpad = -(d + 3) % 8
        table = jnp.concatenate(
            [f, cur_xyz] + ([jnp.zeros((B, N, dpad), jnp.float32)] if dpad else []),
            axis=-1).reshape(B * N, d + 3 + dpad)
        nidx_glob = (nidx + (jnp.arange(B, dtype=jnp.int32) * N)[:, None, None]).reshape(-1)
        rows = _sc_gather(table, nidx_glob).reshape(B, M, k, d + 3 + dpad)
        g_feat = rows[..., :d]
        g_xyz = rows[..., d:d + 3]
        rel = g_xyz - new_xyz[:, :, None, :]
        std = jnp.std(rel, axis=(2, 3), keepdims=True) + 1e-5
        rel = rel / std
        g = jnp.concatenate([g_feat, rel], axis=-1)
        h = jax.nn.relu(jnp.einsum('bmkc,co->bmko', g, Wt))
        h = jnp.max(h, axis=2)
        h = jax.nn.relu(h + jax.nn.relu(jnp.einsum('bmc,co->bmo', h, Wb)))
        cur_xyz, cur_planes, f = new_xyz, new_planes, h
    pooled = jnp.max(f, axis=1)
    return _classifier_head(pooled, Wc1, Wc2, Wc3)
